# baseline (speedup 1.0000x reference)
import functools

import jax
import jax.numpy as jnp
from jax import lax
from jax.experimental import pallas as pl
from jax.experimental.pallas import tpu as pltpu
from jax.experimental.pallas import tpu_sc as plsc

NUM_EMB = 1000000
HALF = NUM_EMB // 2
DIM = 32
ROWS, COLS = 16384, 50
PAD_COLS, PAD_DIM = 56, 128
B = ROWS * COLS
NC, NS = 2, 16
NW = NC * NS
R_PER_W = ROWS // NW
T_PER_C = 32
CHUNK = T_PER_C * COLS
N_CHUNKS = R_PER_W // T_PER_C
L = 16
K_GROUPS = CHUNK // L
SENT = -1


@functools.partial(
    pl.kernel,
    mesh=plsc.VectorSubcoreMesh(core_axis_name="c", subcore_axis_name="s"),
    out_type=jax.ShapeDtypeStruct((ROWS, PAD_COLS, PAD_DIM), jnp.float32),
    compiler_params=pltpu.CompilerParams(use_tc_tiling_on_sc=False),
    scratch_types=[
        pltpu.VMEM((2, CHUNK), jnp.int32),
        pltpu.VMEM((2, 2, CHUNK), jnp.int32),
        pltpu.VMEM((CHUNK, DIM), jnp.float32),
        pltpu.VMEM((CHUNK, DIM), jnp.float32),
        pltpu.SemaphoreType.DMA,
        pltpu.SemaphoreType.DMA,
        pltpu.SemaphoreType.DMA,
        pltpu.SemaphoreType.DMA,
        pltpu.SemaphoreType.DMA,
    ],
)
def _gather_sc(lo_hbm, hi_hbm, idx_hbm, out_hbm,
               idx_v, flt_v, row0, row1,
               sem_i, sem_g0, sem_g1, sem_w0, sem_w1):
    wid = lax.axis_index("s") * NC + lax.axis_index("c")
    base = wid * R_PER_W * COLS
    halves = (lo_hbm, hi_hbm)
    rows = (row0, row1)
    gsems = (sem_g0, sem_g1)
    wsems = (sem_w0, sem_w1)

    def compute_flt(p):
        def body(k, carry):
            ids = idx_v[p, pl.ds(k * L, L)]
            in_hi = ids >= HALF
            sent = jnp.full((L,), SENT, jnp.int32)
            flt_v[p, 0, pl.ds(k * L, L)] = lax.select(in_hi, sent, ids)
            flt_v[p, 1, pl.ds(k * L, L)] = lax.select(in_hi, ids - HALF, sent)
            return carry
        lax.fori_loop(0, K_GROUPS, body, 0)

    def issue_idx(c, p):
        return pltpu.async_copy(
            idx_hbm.at[pl.ds(base + c * CHUNK, CHUNK)], idx_v.at[p], sem_i
        )

    def drain_idx(c, p):
        pltpu.make_async_copy(
            idx_hbm.at[pl.ds(base + c * CHUNK, CHUNK)], idx_v.at[p], sem_i
        ).wait()

    def issue_gathers(p):
        for h in range(2):
            pltpu.async_copy(
                halves[h].at[plsc.Indices(flt_v.at[p, h], ignored_value=SENT)],
                rows[p],
                gsems[p],
            )

    def drain_gathers(p):
        for h in range(2):
            pltpu.make_async_copy(
                halves[h].at[plsc.Indices(flt_v.at[p, h], ignored_value=SENT)],
                rows[p],
                gsems[p],
            ).wait()

    def issue_writes(c, p):
        t0 = wid * R_PER_W + c * T_PER_C
        for j in range(T_PER_C):
            pltpu.async_copy(
                rows[p].at[pl.ds(j * COLS, COLS), :],
                out_hbm.at[t0 + j, pl.ds(0, COLS), pl.ds(0, DIM)],
                wsems[p],
            )

    def drain_writes(c, p):
        t0 = wid * R_PER_W + c * T_PER_C
        for j in range(T_PER_C):
            pltpu.make_async_copy(
                rows[p].at[pl.ds(j * COLS, COLS), :],
                out_hbm.at[t0 + j, pl.ds(0, COLS), pl.ds(0, DIM)],
                wsems[p],
            ).wait()

    pltpu.sync_copy(idx_hbm.at[pl.ds(base, CHUNK)], idx_v.at[0])
    issue_idx(1, 1)
    compute_flt(0)
    issue_gathers(0)

    def subchunk(c, p):
        @pl.when(c + 1 < N_CHUNKS)
        def _():
            drain_idx(c + 1, 1 - p)
            compute_flt(1 - p)
            @pl.when(c >= 1)
            def _():
                drain_writes(c - 1, 1 - p)
            issue_gathers(1 - p)
        drain_gathers(p)
        issue_writes(c, p)
        @pl.when(c + 2 < N_CHUNKS)
        def _():
            issue_idx(c + 2, p)

    def outer(g, carry):
        subchunk(2 * g, 0)
        subchunk(2 * g + 1, 1)
        return carry

    lax.fori_loop(0, N_CHUNKS // 2, outer, 0)
    drain_writes(N_CHUNKS - 2, 0)
    drain_writes(N_CHUNKS - 1, 1)


def kernel(token_ids, weight):
    idx = jnp.minimum(token_ids.reshape(-1), NUM_EMB - 1).astype(jnp.int32)
    out = _gather_sc(weight[:HALF], weight[HALF:], idx)
    return out[:, :COLS, :DIM]
